# baseline (speedup 1.0000x reference)
import jax
import jax.numpy as jnp
from jax.experimental import pallas as pl
from jax.experimental.pallas import tpu as pltpu

HIDDEN_DIM = 768
NUM_EXPERTS = 8
TOP_K = 2
N_TOKENS = 32768

BLOCK = 1024


def _router_kernel(x_ref, hp_ref, idxt_ref):
    x = x_ref[...]
    hp = hp_ref[...]
    scores = jnp.abs(
        jax.lax.dot_general(
            hp, x, (((1,), (1,)), ((), ())),
            preferred_element_type=jnp.float32,
        )
    )
    iota = jax.lax.broadcasted_iota(jnp.int32, scores.shape, 0)
    m1 = jnp.max(scores, axis=0, keepdims=True)
    i1 = jnp.min(jnp.where(scores == m1, iota, NUM_EXPERTS),
                 axis=0, keepdims=True)
    masked = jnp.where(iota == i1, -1.0, scores)
    m2 = jnp.max(masked, axis=0, keepdims=True)
    i2 = jnp.min(jnp.where(masked == m2, iota, NUM_EXPERTS),
                 axis=0, keepdims=True)
    idxt_ref[...] = jnp.concatenate([i1, i2], axis=0)


def kernel(x, hash_planes):
    n = x.shape[0]
    grid = (n // BLOCK,)
    idxt = pl.pallas_call(
        _router_kernel,
        grid=grid,
        in_specs=[
            pl.BlockSpec((BLOCK, HIDDEN_DIM), lambda i: (i, 0)),
            pl.BlockSpec((NUM_EXPERTS, HIDDEN_DIM), lambda i: (0, 0)),
        ],
        out_specs=pl.BlockSpec((TOP_K, BLOCK), lambda i: (0, i)),
        out_shape=jax.ShapeDtypeStruct((TOP_K, n), jnp.int32),
        compiler_params=pltpu.CompilerParams(
            dimension_semantics=("arbitrary",),
        ),
    )(x, hash_planes)
    topk_indices = idxt.T
    topk_probs = jnp.full((n, TOP_K), 1.0 / TOP_K, jnp.float32)
    probs_uniform = jnp.full((n, NUM_EXPERTS), 1.0 / NUM_EXPERTS, jnp.float32)
    return (topk_indices, topk_probs, probs_uniform)


# speedup vs baseline: 1.2568x; 1.2568x over previous
import jax
import jax.numpy as jnp
from jax.experimental import pallas as pl
from jax.experimental.pallas import tpu as pltpu

HIDDEN_DIM = 768
NUM_EXPERTS = 8
TOP_K = 2
N_TOKENS = 32768

BLOCK = 4096


def _router_kernel(x_ref, hp_ref, idxt_ref):
    x = x_ref[...]
    hp = hp_ref[...]
    scores = jnp.abs(
        jax.lax.dot_general(
            hp, x, (((1,), (1,)), ((), ())),
            preferred_element_type=jnp.float32,
        )
    )
    iota = jax.lax.broadcasted_iota(jnp.int32, scores.shape, 0)
    m1 = jnp.max(scores, axis=0, keepdims=True)
    i1 = jnp.min(jnp.where(scores == m1, iota, NUM_EXPERTS),
                 axis=0, keepdims=True)
    masked = jnp.where(iota == i1, -1.0, scores)
    m2 = jnp.max(masked, axis=0, keepdims=True)
    i2 = jnp.min(jnp.where(masked == m2, iota, NUM_EXPERTS),
                 axis=0, keepdims=True)
    idxt_ref[...] = jnp.concatenate([i1, i2], axis=0)


def kernel(x, hash_planes):
    n = x.shape[0]
    grid = (n // BLOCK,)
    idxt = pl.pallas_call(
        _router_kernel,
        grid=grid,
        in_specs=[
            pl.BlockSpec((BLOCK, HIDDEN_DIM), lambda i: (i, 0)),
            pl.BlockSpec((NUM_EXPERTS, HIDDEN_DIM), lambda i: (0, 0)),
        ],
        out_specs=pl.BlockSpec((TOP_K, BLOCK), lambda i: (0, i)),
        out_shape=jax.ShapeDtypeStruct((TOP_K, n), jnp.int32),
        compiler_params=pltpu.CompilerParams(
            dimension_semantics=("parallel",),
        ),
    )(x, hash_planes)
    topk_indices = idxt.T
    topk_probs = jnp.full((n, TOP_K), 1.0 / TOP_K, jnp.float32)
    probs_uniform = jnp.full((n, NUM_EXPERTS), 1.0 / NUM_EXPERTS, jnp.float32)
    return (topk_indices, topk_probs, probs_uniform)
